# Initial kernel scaffold; baseline (speedup 1.0000x reference)
import jax
import jax.numpy as jnp
from jax.experimental import pallas as pl


def kernel(A_z, A_pos, A_batch, G_z, G_pos, G_batch, add_features, emb, Wg1, bg1, Wg2, bg2, Wc1, Wc2, bc2, Wl, bl, Wfc1, bfc1, a_prelu, Wfc2, bfc2):
    raise NotImplementedError("write your pallas kernel here")


# speedup vs baseline: 4.7792x; 4.7792x over previous
import functools

import jax
import jax.numpy as jnp
from jax import lax
from jax.experimental import pallas as pl
from jax.experimental.pallas import tpu as pltpu
from jax.experimental.pallas import tpu_sc as plsc

N = 10000
HIDDEN = 64
NFILT = 64
NINT = 3
NG = 50
CUTOFF = 10.0
MAXNB = 16
NGRAPH = 16

_INF = float("inf")
_BIG = 2**30
_LOG2 = 0.6931471805599453


def _ssp(x):
    return jnp.maximum(x, 0.0) + jnp.log(1.0 + jnp.exp(-jnp.abs(x))) - _LOG2


def _edge_body(posr, posT, batr, batc, c0s, c1s, idx_out, val_out, d2buf,
               *, R, W, n):
    i = pl.program_id(0)
    c0 = c0s[i]
    c1 = c1s[i]
    n2r = jnp.sum(posr[...] * posr[...], axis=1, keepdims=True)
    row_ids = i * R + lax.broadcasted_iota(jnp.int32, (R, 1), 0)
    br = batr[...]

    def fill(c, _):
        colT = posT[:, pl.ds(c * W, W)]
        bc = batc[:, pl.ds(c * W, W)]
        n2c = jnp.sum(colT * colT, axis=0, keepdims=True)
        d2 = n2r + n2c - 2.0 * jnp.dot(posr[...], colT,
                                       preferred_element_type=jnp.float32)
        d2 = jnp.maximum(d2, 0.0)
        col_ids = c * W + lax.broadcasted_iota(jnp.int32, (R, W), 1)
        ok = (br == bc) & (row_ids != col_ids) & (d2 <= CUTOFF * CUTOFF)
        d2buf[:, pl.ds(c * W, W)] = jnp.where(ok, d2, _INF)
        return 0

    lax.fori_loop(c0, c1, fill, 0)

    iota_f = lax.broadcasted_iota(jnp.int32, (R, W), 1).astype(jnp.float32)
    last_v = jnp.full((R, 1), -_INF, jnp.float32)
    last_i = jnp.full((R, 1), -1.0, jnp.float32)
    cols_i = []
    cols_v = []
    for _ in range(MAXNB):
        def sweep(c, acc):
            acc_v, acc_i = acc
            v = d2buf[:, pl.ds(c * W, W)]
            cf = (c * W).astype(jnp.float32) + iota_f
            cand = (v > last_v) | ((v == last_v) & (cf > last_i))
            vm = jnp.where(cand, v, _INF)
            m2 = vm[:, :128]
            i2 = cf[:, :128]
            for k in range(1, W // 128):
                vk = vm[:, k * 128:(k + 1) * 128]
                ik = cf[:, k * 128:(k + 1) * 128]
                take = (vk < m2) | ((vk == m2) & (ik < i2))
                m2 = jnp.where(take, vk, m2)
                i2 = jnp.where(take, ik, i2)
            m = jnp.min(m2, axis=1, keepdims=True)
            im = jnp.min(jnp.where(m2 == m, i2, float(_BIG)), axis=1,
                         keepdims=True)
            better = (m < acc_v) | ((m == acc_v) & (im < acc_i))
            return (jnp.where(better, m, acc_v), jnp.where(better, im, acc_i))

        acc_v, acc_i = lax.fori_loop(
            c0, c1, sweep,
            (jnp.full((R, 1), _INF, jnp.float32),
             jnp.full((R, 1), float(_BIG), jnp.float32)))
        last_v, last_i = acc_v, acc_i
        cols_i.append(jnp.clip(acc_i.astype(jnp.int32), 0, n - 1))
        cols_v.append((acc_v <= CUTOFF * CUTOFF).astype(jnp.float32))
    idx_out[...] = jnp.concatenate(cols_i, axis=1)
    val_out[...] = jnp.concatenate(cols_v, axis=1)


def _build_edges(pos8, posT, batT, batC, c0s, c1s, *, n, R, W, nch):
    nt = n // R
    body = functools.partial(_edge_body, R=R, W=W, n=n)
    return pl.pallas_call(
        body,
        grid=(nt,),
        in_specs=[
            pl.BlockSpec((R, 8), lambda i: (i, 0)),
            pl.BlockSpec((8, nch * W), lambda i: (0, 0)),
            pl.BlockSpec((R, 1), lambda i: (i, 0)),
            pl.BlockSpec((1, nch * W), lambda i: (0, 0)),
            pl.BlockSpec(memory_space=pltpu.SMEM),
            pl.BlockSpec(memory_space=pltpu.SMEM),
        ],
        out_specs=[
            pl.BlockSpec((R, MAXNB), lambda i: (i, 0)),
            pl.BlockSpec((R, MAXNB), lambda i: (i, 0)),
        ],
        out_shape=[
            jax.ShapeDtypeStruct((n, MAXNB), jnp.int32),
            jax.ShapeDtypeStruct((n, MAXNB), jnp.float32),
        ],
        scratch_shapes=[pltpu.VMEM((R, nch * W), jnp.float32)],
    )(pos8, posT, batT, batC, c0s, c1s)


def _sc_gather(table, idx, req_chunk):
    B = idx.shape[0]
    D = table.shape[1]
    b_per_w = B // 32
    chunk = 8
    for cand in range(8, min(req_chunk, b_per_w) + 1, 8):
        if b_per_w % cand == 0:
            chunk = cand
    npairs = b_per_w // (2 * chunk) if (b_per_w // chunk) % 2 == 0 else 0
    mesh = plsc.VectorSubcoreMesh(core_axis_name="c", subcore_axis_name="s")

    @functools.partial(
        pl.kernel, mesh=mesh,
        compiler_params=pltpu.CompilerParams(use_tc_tiling_on_sc=False),
        out_type=jax.ShapeDtypeStruct((B, D), jnp.float32),
        scratch_types=[
            pltpu.VMEM((chunk,), jnp.int32),
            pltpu.VMEM((chunk,), jnp.int32),
            pltpu.VMEM((chunk, D), jnp.float32),
            pltpu.VMEM((chunk, D), jnp.float32),
            pltpu.SemaphoreType.DMA,
            pltpu.SemaphoreType.DMA,
            pltpu.SemaphoreType.DMA,
            pltpu.SemaphoreType.DMA,
            pltpu.SemaphoreType.DMA,
            pltpu.SemaphoreType.DMA,
        ],
    )
    def k(table_hbm, idx_hbm, out_hbm, idx0, idx1, rows0, rows1,
          si0, si1, sg0, sg1, so0, so1):
        wid = lax.axis_index("s") * 2 + lax.axis_index("c")
        base = wid * b_per_w

        def pair(p, _):
            off0 = base + 2 * p * chunk
            off1 = off0 + chunk
            ci0 = pltpu.async_copy(idx_hbm.at[pl.ds(off0, chunk)], idx0, si0)
            ci1 = pltpu.async_copy(idx_hbm.at[pl.ds(off1, chunk)], idx1, si1)
            ci0.wait()
            g0 = pltpu.async_copy(table_hbm.at[idx0], rows0, sg0)
            ci1.wait()
            g1 = pltpu.async_copy(table_hbm.at[idx1], rows1, sg1)
            g0.wait()
            o0 = pltpu.async_copy(rows0, out_hbm.at[pl.ds(off0, chunk)], so0)
            g1.wait()
            o1 = pltpu.async_copy(rows1, out_hbm.at[pl.ds(off1, chunk)], so1)
            o0.wait()
            o1.wait()
            return 0

        def step(j, _):
            off = base + j * chunk
            pltpu.async_copy(idx_hbm.at[pl.ds(off, chunk)], idx0, si0).wait()
            pltpu.async_copy(table_hbm.at[idx0], rows0, sg0).wait()
            pltpu.async_copy(rows0, out_hbm.at[pl.ds(off, chunk)], so0).wait()
            return 0

        if npairs:
            lax.fori_loop(0, npairs, pair, 0)
        else:
            lax.fori_loop(0, b_per_w // chunk, step, 0)

    return k(table, idx)


def _msg_body(gpos, dpos, g, vld, Wg1, bg1, Wg2, bg2, offs, P, out, *, E_T):
    nd = E_T // MAXNB
    dp = jnp.broadcast_to(dpos[...].reshape(nd, 1, 16), (nd, MAXNB, 16))
    diff = gpos[...] - dp.reshape(E_T, 16)
    d2 = jnp.sum(diff * diff, axis=1, keepdims=True)
    valid = vld[...]
    d = jnp.broadcast_to(jnp.sqrt(jnp.where(valid > 0, d2, 1.0)), (E_T, 64))
    vw = jnp.broadcast_to(valid, (E_T, 64))
    coeff = -0.5 / (CUTOFF / (NG - 1)) ** 2
    rbf = jnp.exp(coeff * (d - offs[...]) ** 2)
    t1 = _ssp(jnp.dot(rbf, Wg1[...], preferred_element_type=jnp.float32)
              + bg1[...])
    Wf = jnp.dot(t1, Wg2[...], preferred_element_type=jnp.float32) + bg2[...]
    Cenv = 0.5 * (jnp.cos(d * (jnp.pi / CUTOFF)) + 1.0) * vw
    m = g[...] * Wf * Cenv
    out[...] = jnp.dot(P[...], m, preferred_element_type=jnp.float32)


def _messages(gpos, pos16, g, vld, Wg1, bg1, Wg2, bg2, offs, P, *, n, E_T):
    E = n * MAXNB
    nt = E // E_T
    nd = E_T // MAXNB
    body = functools.partial(_msg_body, E_T=E_T)
    cst = lambda i: (0, 0)
    return pl.pallas_call(
        body,
        grid=(nt,),
        in_specs=[
            pl.BlockSpec((E_T, 16), lambda i: (i, 0)),
            pl.BlockSpec((nd, 16), lambda i: (i, 0)),
            pl.BlockSpec((E_T, HIDDEN), lambda i: (i, 0)),
            pl.BlockSpec((E_T, 1), lambda i: (i, 0)),
            pl.BlockSpec((64, NFILT), cst),
            pl.BlockSpec((1, NFILT), cst),
            pl.BlockSpec((NFILT, NFILT), cst),
            pl.BlockSpec((1, NFILT), cst),
            pl.BlockSpec((1, 64), cst),
            pl.BlockSpec((nd, E_T), cst),
        ],
        out_specs=pl.BlockSpec((nd, HIDDEN), lambda i: (i, 0)),
        out_shape=jax.ShapeDtypeStruct((n, HIDDEN), jnp.float32),
    )(gpos, pos16, g, vld, Wg1, bg1, Wg2, bg2, offs, P)


def _upd_body(agg, h, Wc2, bc2, Wl, bl, Wc1n, hout, xout):
    x = _ssp(jnp.dot(agg[...], Wc2[...], preferred_element_type=jnp.float32)
             + bc2[...])
    x = jnp.dot(x, Wl[...], preferred_element_type=jnp.float32) + bl[...]
    hn = h[...] + x
    hout[...] = hn
    xout[...] = jnp.dot(hn, Wc1n[...], preferred_element_type=jnp.float32)


def _upd_pool_body(agg, h, Wc2, bc2, Wl, bl, pout):
    i = pl.program_id(0)
    x = _ssp(jnp.dot(agg[...], Wc2[...], preferred_element_type=jnp.float32)
             + bc2[...])
    x = jnp.dot(x, Wl[...], preferred_element_type=jnp.float32) + bl[...]
    hn = h[...] + x

    @pl.when(i == 0)
    def _():
        pout[...] = jnp.zeros_like(pout)

    pout[...] += jnp.sum(hn, axis=0, keepdims=True) / NGRAPH


def _update(agg, h, Wc2, bc2, Wl, bl, Wc1n, *, n, R):
    nt = n // R
    cst = lambda i: (0, 0)
    wspec = [pl.BlockSpec((HIDDEN, HIDDEN), cst), pl.BlockSpec((1, HIDDEN), cst),
             pl.BlockSpec((HIDDEN, HIDDEN), cst), pl.BlockSpec((1, HIDDEN), cst)]
    io = [pl.BlockSpec((R, HIDDEN), lambda i: (i, 0)),
          pl.BlockSpec((R, HIDDEN), lambda i: (i, 0))]
    if Wc1n is not None:
        return pl.pallas_call(
            _upd_body, grid=(nt,),
            in_specs=io + wspec + [pl.BlockSpec((HIDDEN, HIDDEN), cst)],
            out_specs=io,
            out_shape=[jax.ShapeDtypeStruct((n, HIDDEN), jnp.float32)] * 2,
        )(agg, h, Wc2, bc2, Wl, bl, Wc1n)
    return pl.pallas_call(
        _upd_pool_body, grid=(nt,),
        in_specs=io + wspec,
        out_specs=pl.BlockSpec((1, HIDDEN), cst),
        out_shape=jax.ShapeDtypeStruct((1, HIDDEN), jnp.float32),
    )(agg, h, Wc2, bc2, Wl, bl)


def _mm_body(x, W, o):
    o[...] = jnp.dot(x[...], W[...], preferred_element_type=jnp.float32)


def _mm(x, W, *, n, R):
    return pl.pallas_call(
        _mm_body, grid=(n // R,),
        in_specs=[pl.BlockSpec((R, HIDDEN), lambda i: (i, 0)),
                  pl.BlockSpec((HIDDEN, HIDDEN), lambda i: (0, 0))],
        out_specs=pl.BlockSpec((R, HIDDEN), lambda i: (i, 0)),
        out_shape=jax.ShapeDtypeStruct((n, HIDDEN), jnp.float32),
    )(x, W)


def _head_body(a, g, add, W1a, W1g, W1d, b1, ap, W2, b2, o):
    x = (jnp.dot(a[...], W1a[...], preferred_element_type=jnp.float32)
         + jnp.dot(g[...], W1g[...], preferred_element_type=jnp.float32)
         + jnp.dot(add[...], W1d[...], preferred_element_type=jnp.float32)
         + b1[...])
    x = jnp.where(x >= 0, x, ap[...] * x)
    y = jnp.dot(x, W2[...], preferred_element_type=jnp.float32) + b2[...]
    o[...] = jnp.exp(y)


def _head(a, g, add, W1a, W1g, W1d, b1, ap, W2, b2):
    return pl.pallas_call(
        _head_body,
        out_shape=jax.ShapeDtypeStruct((1, 1), jnp.float32),
    )(a, g, add, W1a, W1g, W1d, b1, ap, W2, b2)


def _embed(z, pos, batch, emb, Wg1, bg1, Wg2, bg2, Wc1, Wc2, bc2, Wl, bl,
           offs):
    n = pos.shape[0]
    R = 200 if n % 200 == 0 else 64
    W = 256
    ncols = ((n + W - 1) // W) * W
    nch = ncols // W
    pos8 = jnp.pad(pos, ((0, 0), (0, 5)))
    pos16 = jnp.pad(pos, ((0, 0), (0, 13)))
    posT = jnp.pad(pos8.T, ((0, 0), (0, ncols - n)))
    batch = batch.astype(jnp.int32)
    batC = jnp.pad(batch, (0, ncols - n), constant_values=_BIG).reshape(1, -1)
    batT = batch.reshape(-1, 1)
    ch_lo = batC[0, ::W]
    ch_hi = batC[0, W - 1::W]
    r_lo = batch[::R]
    r_hi = batch[R - 1::R]
    c0s = jnp.sum((ch_hi[None, :] < r_lo[:, None]).astype(jnp.int32), axis=1)
    c1s = nch - jnp.sum((ch_lo[None, :] > r_hi[:, None]).astype(jnp.int32),
                        axis=1)

    idx, vld = _build_edges(pos8, posT, batT, batC, c0s, c1s,
                            n=n, R=R, W=W, nch=nch)
    E = n * MAXNB
    Ep = ((E + 4095) // 4096) * 4096
    src = jnp.pad(idx.reshape(E), (0, Ep - E))
    vldf = vld.reshape(E, 1)

    gpos = _sc_gather(pos16, src, 128)
    h = _sc_gather(emb, jnp.pad(z.astype(jnp.int32), (0, 256 - n % 256)),
                   64)[:n] if n % 256 else _sc_gather(emb, z.astype(jnp.int32),
                                                      64)
    Rn = 2000 if n % 2000 == 0 else 8
    xh = _mm(h, Wc1[0], n=n, R=Rn)
    E_T = 640 if (n * MAXNB) % 640 == 0 else 256
    Pmat = (lax.broadcasted_iota(jnp.int32, (E_T // MAXNB, E_T), 1) // MAXNB
            == lax.broadcasted_iota(jnp.int32, (E_T // MAXNB, E_T), 0)
            ).astype(jnp.float32)
    for t in range(NINT):
        g = _sc_gather(xh, src, 128)
        agg = _messages(gpos, pos16, g, vldf, Wg1[t], bg1[t:t + 1], Wg2[t],
                        bg2[t:t + 1], offs, Pmat, n=n, E_T=E_T)
        nxt = Wc1[t + 1] if t + 1 < NINT else None
        res = _update(agg, h, Wc2[t], bc2[t:t + 1], Wl[t], bl[t:t + 1], nxt,
                      n=n, R=Rn)
        if nxt is not None:
            h, xh = res
        else:
            pooled = res
    return pooled


def kernel(A_z, A_pos, A_batch, G_z, G_pos, G_batch, add_features, emb, Wg1,
           bg1, Wg2, bg2, Wc1, Wc2, bc2, Wl, bl, Wfc1, bfc1, a_prelu, Wfc2,
           bfc2):
    offs = jnp.pad(jnp.linspace(0.0, CUTOFF, NG), (0, 64 - NG)).reshape(1, 64)
    Wg1p = jnp.pad(Wg1, ((0, 0), (0, 64 - NG), (0, 0)))
    args = (emb, Wg1p, bg1, Wg2, bg2, Wc1, Wc2, bc2, Wl, bl, offs)
    Aemb = _embed(A_z, A_pos, A_batch, *args)
    Gemb = _embed(G_z, G_pos, G_batch, *args)
    out = _head(Aemb, Gemb, add_features.reshape(1, -1),
                Wfc1[:HIDDEN], Wfc1[HIDDEN:2 * HIDDEN], Wfc1[2 * HIDDEN:],
                bfc1.reshape(1, -1), a_prelu.reshape(1, 1), Wfc2,
                bfc2.reshape(1, 1))
    return out.reshape(1)
